# baseline (speedup 1.0000x reference)
import functools

import jax
import jax.numpy as jnp
from jax import lax
from jax.experimental import pallas as pl
from jax.experimental.pallas import tpu as pltpu
from jax.experimental.pallas import tpu_sc as plsc

N = 4096
C = 256
NV = 3
BQ = 128
NB = N // BQ
CK = 128
NCHK = N // CK
B_OUT = 4 * N

NC = 2
NS = 16
NW = NC * NS
BPW = B_OUT // NW
CH = 128
NCH = BPW // CH


def _argmin_body(params_ref, qx_ref, qy_ref, cx_ref, cy_ref, out_ref):
    v = pl.program_id(0)
    ax = params_ref[v]
    ay = params_ref[NV + v]
    s0 = params_ref[2 * NV]
    s1 = params_ref[2 * NV + 1]
    qx = (qx_ref[...] + ax) - s0
    qy = (qy_ref[...] + ay) - s1
    lane = lax.broadcasted_iota(jnp.int32, (BQ, CK), 1)

    def chunk_body(c, carry):
        accv, acci = carry
        cxc = cx_ref[pl.ds(c, 1), :]
        cyc = cy_ref[pl.ds(c, 1), :]
        dx = qx - cxc
        dy = qy - cyc
        dist = jnp.sqrt(dx * dx + dy * dy)
        j = lane + c * CK
        upd = dist < accv
        accv = jnp.where(upd, dist, accv)
        acci = jnp.where(upd, j, acci)
        return accv, acci

    accv0 = jnp.full((BQ, CK), jnp.inf, jnp.float32)
    acci0 = jnp.zeros((BQ, CK), jnp.int32)
    accv, acci = lax.fori_loop(0, NCHK, chunk_body, (accv0, acci0))
    m = jnp.min(accv, axis=1, keepdims=True)
    idx = jnp.min(jnp.where(accv == m, acci, N), axis=1)
    out_ref[0, 0, :] = idx


_argmin_call = pl.pallas_call(
    _argmin_body,
    grid=(NV, NB),
    in_specs=[
        pl.BlockSpec(memory_space=pltpu.SMEM),
        pl.BlockSpec((BQ, 1), lambda v, b: (b, 0)),
        pl.BlockSpec((BQ, 1), lambda v, b: (b, 0)),
        pl.BlockSpec((NCHK, CK), lambda v, b: (0, 0)),
        pl.BlockSpec((NCHK, CK), lambda v, b: (0, 0)),
    ],
    out_specs=pl.BlockSpec((1, 1, BQ), lambda v, b: (v * NB + b, 0, 0)),
    out_shape=jax.ShapeDtypeStruct((NV * NB, 1, BQ), jnp.int32),
)


@functools.lru_cache(maxsize=1)
def _make_sc_gather():
    mesh = plsc.VectorSubcoreMesh(core_axis_name="c", subcore_axis_name="s")

    @functools.partial(
        pl.kernel,
        mesh=mesh,
        out_type=jax.ShapeDtypeStruct((B_OUT, C), jnp.float32),
        scratch_types=[
            pltpu.VMEM((BPW,), jnp.int32),
            pltpu.VMEM((CH, C), jnp.float32),
            pltpu.SemaphoreType.DMA,
        ],
    )
    def _sc_gather(table_hbm, idx_hbm, out_hbm, idx_v, rows_v, sem):
        wid = lax.axis_index("s") * NC + lax.axis_index("c")
        base = wid * BPW
        pltpu.sync_copy(idx_hbm.at[pl.ds(base, BPW)], idx_v)
        for c in range(NCH):
            pltpu.async_copy(
                table_hbm.at[idx_v.at[pl.ds(c * CH, CH)]], rows_v, sem
            ).wait()
            pltpu.sync_copy(rows_v, out_hbm.at[pl.ds(base + c * CH, CH)])

    return _sc_gather


def kernel(values, coords, spacing, shift):
    zero = jnp.zeros((), jnp.float32)
    ax = jnp.stack([spacing[0], zero, spacing[0]])
    ay = jnp.stack([spacing[1], spacing[1], zero])
    params = jnp.concatenate([ax, ay, shift.astype(jnp.float32)])
    qx = coords[:, 0:1]
    qy = coords[:, 1:2]
    cx = coords[:, 0].reshape(NCHK, CK)
    cy = coords[:, 1].reshape(NCHK, CK)
    idx = _argmin_call(params, qx, qy, cx, cy).reshape(NV * N)
    allidx = jnp.concatenate([jnp.arange(N, dtype=jnp.int32), idx])
    return _make_sc_gather()(values, allidx)


# speedup vs baseline: 2.6058x; 2.6058x over previous
import functools

import jax
import jax.numpy as jnp
from jax import lax
from jax.experimental import pallas as pl
from jax.experimental.pallas import tpu as pltpu
from jax.experimental.pallas import tpu_sc as plsc

N = 4096
C = 256
NV = 3
BQ = 128
NB = N // BQ
CK = 128
NCHK = N // CK
B_OUT = 4 * N

NC = 2
NS = 16
NW = NC * NS
BPW = B_OUT // NW
CH = 128
NCH = BPW // CH


def _argmin_body(params_ref, qx_ref, qy_ref, cx_ref, cy_ref, out_ref):
    v = pl.program_id(0)
    ax = params_ref[v]
    ay = params_ref[NV + v]
    s0 = params_ref[2 * NV]
    s1 = params_ref[2 * NV + 1]
    qx = (qx_ref[...] + ax) - s0
    qy = (qy_ref[...] + ay) - s1
    lane = lax.broadcasted_iota(jnp.int32, (BQ, CK), 1)

    accv = jnp.full((BQ, CK), jnp.inf, jnp.float32)
    acci = jnp.zeros((BQ, CK), jnp.int32)
    for c in range(NCHK):
        cxc = cx_ref[c:c + 1, :]
        cyc = cy_ref[c:c + 1, :]
        dx = qx - cxc
        dy = qy - cyc
        dist = jnp.sqrt(dx * dx + dy * dy)
        j = lane + c * CK
        upd = dist < accv
        accv = jnp.where(upd, dist, accv)
        acci = jnp.where(upd, j, acci)
    m = jnp.min(accv, axis=1, keepdims=True)
    idx = jnp.min(jnp.where(accv == m, acci, N), axis=1)
    out_ref[0, 0, :] = idx


_argmin_call = pl.pallas_call(
    _argmin_body,
    grid=(NV, NB),
    in_specs=[
        pl.BlockSpec(memory_space=pltpu.SMEM),
        pl.BlockSpec((BQ, 1), lambda v, b: (b, 0)),
        pl.BlockSpec((BQ, 1), lambda v, b: (b, 0)),
        pl.BlockSpec((NCHK, CK), lambda v, b: (0, 0)),
        pl.BlockSpec((NCHK, CK), lambda v, b: (0, 0)),
    ],
    out_specs=pl.BlockSpec((1, 1, BQ), lambda v, b: (v * NB + b, 0, 0)),
    out_shape=jax.ShapeDtypeStruct((NV * NB, 1, BQ), jnp.int32),
)


@functools.lru_cache(maxsize=1)
def _make_sc_gather():
    mesh = plsc.VectorSubcoreMesh(core_axis_name="c", subcore_axis_name="s")

    @functools.partial(
        pl.kernel,
        mesh=mesh,
        out_type=jax.ShapeDtypeStruct((B_OUT, C), jnp.float32),
        scratch_types=[
            pltpu.VMEM((BPW,), jnp.int32),
            pltpu.VMEM((CH, C), jnp.float32),
            pltpu.SemaphoreType.DMA,
        ],
    )
    def _sc_gather(table_hbm, idx_hbm, out_hbm, idx_v, rows_v, sem):
        wid = lax.axis_index("s") * NC + lax.axis_index("c")
        base = wid * BPW
        pltpu.sync_copy(idx_hbm.at[pl.ds(base, BPW)], idx_v)
        for c in range(NCH):
            pltpu.async_copy(
                table_hbm.at[idx_v.at[pl.ds(c * CH, CH)]], rows_v, sem
            ).wait()
            pltpu.sync_copy(rows_v, out_hbm.at[pl.ds(base + c * CH, CH)])

    return _sc_gather


def kernel(values, coords, spacing, shift):
    zero = jnp.zeros((), jnp.float32)
    ax = jnp.stack([spacing[0], zero, spacing[0]])
    ay = jnp.stack([spacing[1], spacing[1], zero])
    params = jnp.concatenate([ax, ay, shift.astype(jnp.float32)])
    qx = coords[:, 0:1]
    qy = coords[:, 1:2]
    cx = coords[:, 0].reshape(NCHK, CK)
    cy = coords[:, 1].reshape(NCHK, CK)
    idx = _argmin_call(params, qx, qy, cx, cy).reshape(NV * N)
    allidx = jnp.concatenate([jnp.arange(N, dtype=jnp.int32), idx])
    return _make_sc_gather()(values, allidx)


# speedup vs baseline: 3.5240x; 1.3524x over previous
import functools

import jax
import jax.numpy as jnp
from jax import lax
from jax.experimental import pallas as pl
from jax.experimental.pallas import tpu as pltpu
from jax.experimental.pallas import tpu_sc as plsc

N = 4096
C = 256
NV = 3
BQ = 128
NB = N // BQ
CK = 128
NCHK = N // CK
B_OUT = 4 * N

NC = 2
NS = 16
NW = NC * NS
BPW = B_OUT // NW
CH = 128
NCH = BPW // CH


def _sqrt_pos(s):
    return jnp.where(s == 0.0, jnp.float32(0.0), s * lax.rsqrt(s))


def _argmin_body(params_ref, qx_ref, qy_ref, cx_ref, cy_ref, out_ref):
    sp0 = params_ref[0]
    sp1 = params_ref[1]
    s0 = params_ref[2]
    s1 = params_ref[3]
    qx0 = (qx_ref[...] + sp0) - s0
    qxp = qx_ref[...] - s0
    qy1 = (qy_ref[...] + sp1) - s1
    qyp = qy_ref[...] - s1
    lane = lax.broadcasted_iota(jnp.int32, (BQ, CK), 1)

    inf = jnp.full((BQ, CK), jnp.inf, jnp.float32)
    zero = jnp.zeros((BQ, CK), jnp.int32)
    accv = [inf, inf, inf]
    acci = [zero, zero, zero]
    for c in range(NCHK):
        cxc = cx_ref[c:c + 1, :]
        cyc = cy_ref[c:c + 1, :]
        a = qx0 - cxc
        b = qy1 - cyc
        cc = qxp - cxc
        d = qyp - cyc
        a2 = a * a
        b2 = b * b
        dists = (
            _sqrt_pos(a2 + b2),
            _sqrt_pos(cc * cc + b2),
            _sqrt_pos(a2 + d * d),
        )
        j = lane + c * CK
        for v in range(NV):
            upd = dists[v] < accv[v]
            accv[v] = jnp.where(upd, dists[v], accv[v])
            acci[v] = jnp.where(upd, j, acci[v])
    for v in range(NV):
        m = jnp.min(accv[v], axis=1, keepdims=True)
        idx = jnp.min(jnp.where(accv[v] == m, acci[v], N), axis=1)
        out_ref[v, 0, 0, :] = idx


_argmin_call = pl.pallas_call(
    _argmin_body,
    grid=(NB,),
    in_specs=[
        pl.BlockSpec(memory_space=pltpu.SMEM),
        pl.BlockSpec((BQ, 1), lambda b: (b, 0)),
        pl.BlockSpec((BQ, 1), lambda b: (b, 0)),
        pl.BlockSpec((NCHK, CK), lambda b: (0, 0)),
        pl.BlockSpec((NCHK, CK), lambda b: (0, 0)),
    ],
    out_specs=pl.BlockSpec((NV, 1, 1, BQ), lambda b: (0, b, 0, 0)),
    out_shape=jax.ShapeDtypeStruct((NV, NB, 1, BQ), jnp.int32),
)


@functools.lru_cache(maxsize=1)
def _make_sc_gather():
    mesh = plsc.VectorSubcoreMesh(core_axis_name="c", subcore_axis_name="s")

    @functools.partial(
        pl.kernel,
        mesh=mesh,
        out_type=jax.ShapeDtypeStruct((B_OUT, C), jnp.float32),
        scratch_types=[
            pltpu.VMEM((BPW,), jnp.int32),
            pltpu.VMEM((CH, C), jnp.float32),
            pltpu.SemaphoreType.DMA,
        ],
    )
    def _sc_gather(table_hbm, idx_hbm, out_hbm, idx_v, rows_v, sem):
        wid = lax.axis_index("s") * NC + lax.axis_index("c")
        base = wid * BPW
        pltpu.sync_copy(idx_hbm.at[pl.ds(base, BPW)], idx_v)
        for c in range(NCH):
            pltpu.async_copy(
                table_hbm.at[idx_v.at[pl.ds(c * CH, CH)]], rows_v, sem
            ).wait()
            pltpu.sync_copy(rows_v, out_hbm.at[pl.ds(base + c * CH, CH)])

    return _sc_gather


def kernel(values, coords, spacing, shift):
    params = jnp.concatenate(
        [spacing.astype(jnp.float32), shift.astype(jnp.float32)]
    )
    qx = coords[:, 0:1]
    qy = coords[:, 1:2]
    cx = coords[:, 0].reshape(NCHK, CK)
    cy = coords[:, 1].reshape(NCHK, CK)
    idx = _argmin_call(params, qx, qy, cx, cy).reshape(NV * N)
    allidx = jnp.concatenate([jnp.arange(N, dtype=jnp.int32), idx])
    return _make_sc_gather()(values, allidx)
